# baseline (speedup 1.0000x reference)
import jax
import jax.numpy as jnp
from jax import lax
from jax.experimental import pallas as pl
from jax.experimental.pallas import tpu as pltpu
from jax.experimental.pallas import tpu_sc as plsc

N_LANES = 16
NUM_CORES = 2
NUM_SUBCORES = 16
NUM_WORKERS = NUM_CORES * NUM_SUBCORES
BATCH = 16384
FACTORS = 64
ROWS_PER_WORKER = BATCH // NUM_WORKERS
GROUPS = ROWS_PER_WORKER // N_LANES


def _mf_body(uid_hbm, iid_hbm, p_hbm, q_hbm, bu_hbm, bi_hbm, out_hbm,
             uidx_v, iidx_v, prow_v, qrow_v, bu_v, bi_v, out_v, stage_v, sem):
    wid = lax.axis_index("s") * NUM_CORES + lax.axis_index("c")
    base = wid * ROWS_PER_WORKER

    pltpu.sync_copy(uid_hbm.at[pl.ds(base, ROWS_PER_WORKER)], uidx_v)
    pltpu.sync_copy(iid_hbm.at[pl.ds(base, ROWS_PER_WORKER)], iidx_v)

    cp_p = pltpu.async_copy(p_hbm.at[uidx_v], prow_v, sem)
    cp_q = pltpu.async_copy(q_hbm.at[iidx_v], qrow_v, sem)
    cp_bu = pltpu.async_copy(bu_hbm.at[uidx_v], bu_v, sem)
    cp_bi = pltpu.async_copy(bi_hbm.at[iidx_v], bi_v, sem)
    cp_p.wait()
    cp_q.wait()
    cp_bu.wait()
    cp_bi.wait()

    lane = lax.iota(jnp.int32, N_LANES)

    def group(g, _):
        for l in range(N_LANES):
            r = g * N_LANES + l
            v = prow_v[r, pl.ds(0, 16)] * qrow_v[r, pl.ds(0, 16)]
            for j in range(1, FACTORS // N_LANES):
                v = v + prow_v[r, pl.ds(j * 16, 16)] * qrow_v[r, pl.ds(j * 16, 16)]
            stage_v[pl.ds(l * N_LANES, N_LANES)] = v
        acc = bu_v[pl.ds(g * N_LANES, N_LANES)] + bi_v[pl.ds(g * N_LANES, N_LANES)]
        for c in range(N_LANES):
            acc = acc + plsc.load_gather(stage_v, [lane * N_LANES + c])
        out_v[pl.ds(g * N_LANES, N_LANES)] = acc
        return None

    lax.fori_loop(0, GROUPS, group, None)
    pltpu.sync_copy(out_v, out_hbm.at[pl.ds(base, ROWS_PER_WORKER)])


@jax.jit
def kernel(user_id, item_id, P, Q, user_bias, item_bias):
    mesh = plsc.VectorSubcoreMesh(
        core_axis_name="c", subcore_axis_name="s",
        num_cores=NUM_CORES, num_subcores=NUM_SUBCORES)
    run = pl.kernel(
        _mf_body,
        out_type=jax.ShapeDtypeStruct((BATCH,), jnp.float32),
        mesh=mesh,
        scratch_types=[
            pltpu.VMEM((ROWS_PER_WORKER,), jnp.int32),
            pltpu.VMEM((ROWS_PER_WORKER,), jnp.int32),
            pltpu.VMEM((ROWS_PER_WORKER, FACTORS), jnp.float32),
            pltpu.VMEM((ROWS_PER_WORKER, FACTORS), jnp.float32),
            pltpu.VMEM((ROWS_PER_WORKER,), jnp.float32),
            pltpu.VMEM((ROWS_PER_WORKER,), jnp.float32),
            pltpu.VMEM((ROWS_PER_WORKER,), jnp.float32),
            pltpu.VMEM((N_LANES * N_LANES,), jnp.float32),
            pltpu.SemaphoreType.DMA,
        ],
        compiler_params=pltpu.CompilerParams(
            needs_layout_passes=False, use_tc_tiling_on_sc=False),
    )
    return run(user_id.astype(jnp.int32), item_id.astype(jnp.int32),
               P, Q, user_bias.reshape(-1), item_bias.reshape(-1))
